# baseline (speedup 1.0000x reference)
import functools

import jax
import jax.numpy as jnp
from jax import lax
from jax.experimental import pallas as pl
from jax.experimental.pallas import tpu as pltpu
from jax.experimental.pallas import tpu_sc as plsc

_N = 100000
_NPAD = 100096
_W = 16
_NC = 2
_NS = 16
_NW = _NC * _NS
_KB = 8
_CH = _KB * 128
_STRIPE = _NPAD // _NS
_R = 256


def _mesh():
    return plsc.VectorSubcoreMesh(
        core_axis_name="c", subcore_axis_name="s", num_cores=_NC, num_subcores=_NS
    )


_SC_PARAMS = pltpu.CompilerParams(use_tc_tiling_on_sc=False)


_DW = 8


def _sc_degree(dsts, ones, zeros_col):
    T = dsts.shape[1]

    @functools.partial(
        pl.kernel,
        out_type=jax.ShapeDtypeStruct((_NC, _NPAD, _DW), jnp.float32),
        mesh=_mesh(),
        scratch_types=[
            pltpu.VMEM((2, _KB, 128), jnp.int32),
            pltpu.VMEM((128, _DW), jnp.float32),
            pltpu.VMEM((_STRIPE // 8, _DW), jnp.float32),
            pltpu.VMEM_SHARED((_NPAD, _DW), jnp.float32),
            pltpu.SemaphoreType.DMA,
            pltpu.SemaphoreType.DMA,
        ],
        compiler_params=_SC_PARAMS,
    )
    def k(dsts_hbm, ones_hbm, z_hbm, out_hbm, idx_d, ones_v, zbuf, acc, isem, ssem):
        cid = lax.axis_index("c")
        sid = lax.axis_index("s")
        wid = cid * _NS + sid
        pltpu.sync_copy(ones_hbm, ones_v)
        pltpu.sync_copy(z_hbm, zbuf)
        for q in range(8):
            pltpu.sync_copy(
                zbuf, acc.at[pl.ds(sid * _STRIPE + q * (_STRIPE // 8), _STRIPE // 8)]
            )
        plsc.subcore_barrier()
        pltpu.async_copy(dsts_hbm.at[wid, 0], idx_d.at[0], isem)

        def chunk(c, p, first, last):
            pltpu.make_async_copy(dsts_hbm.at[wid, c], idx_d.at[p], isem).wait()
            for j in range(_KB):
                if first is None:
                    pltpu.make_async_copy(
                        ones_v, acc.at[idx_d.at[1 - p].at[j]], ssem
                    ).wait()
                else:
                    @pl.when(first)
                    def _(j=j):
                        pltpu.make_async_copy(
                            ones_v, acc.at[idx_d.at[1 - p].at[j]], ssem
                        ).wait()

            @pl.when(last)
            def _():
                pltpu.async_copy(dsts_hbm.at[wid, c + 1], idx_d.at[1 - p], isem)

            for j in range(_KB):
                pltpu.async_copy(ones_v, acc.at[idx_d.at[p].at[j]], ssem, add=True)

        def body(u, carry):
            a = 2 * u
            chunk(a, 0, u > 0, a + 1 < T)
            chunk(a + 1, 1, None, a + 2 < T)
            return carry

        lax.fori_loop(0, T // 2, body, 0)
        for j in range(_KB):
            pltpu.make_async_copy(ones_v, acc.at[idx_d.at[1].at[j]], ssem).wait()
        plsc.subcore_barrier()
        for q in range(8):
            off = sid * _STRIPE + q * (_STRIPE // 8)
            pltpu.sync_copy(acc.at[pl.ds(off, _STRIPE // 8)], zbuf)
            pltpu.sync_copy(zbuf, out_hbm.at[cid, pl.ds(off, _STRIPE // 8)])

    return k(dsts, ones, zeros_col)


def _sc_pass(srcs, dsts, table, zeros_tbl):
    T = srcs.shape[1]

    @functools.partial(
        pl.kernel,
        out_type=jax.ShapeDtypeStruct((_NC, _NPAD, _W), jnp.float32),
        mesh=_mesh(),
        scratch_types=[
            pltpu.VMEM((2, _KB, 128), jnp.int32),
            pltpu.VMEM((2, _KB, 128), jnp.int32),
            pltpu.VMEM((_KB, 128, _W), jnp.float32),
            pltpu.VMEM((_STRIPE // 16, _W), jnp.float32),
            pltpu.VMEM_SHARED((_NPAD, _W), jnp.float32),
            pltpu.SemaphoreType.DMA,
            pltpu.SemaphoreType.DMA,
            pltpu.SemaphoreType.DMA,
        ],
        compiler_params=_SC_PARAMS,
    )
    def k(srcs_hbm, dsts_hbm, tbl_hbm, z_hbm, out_hbm,
          idx_s, idx_d, rows, zbuf, acc, isem, gsem, ssem):
        cid = lax.axis_index("c")
        sid = lax.axis_index("s")
        wid = cid * _NS + sid
        zrows = _STRIPE // 16
        pltpu.sync_copy(z_hbm, zbuf)
        for q in range(16):
            pltpu.sync_copy(zbuf, acc.at[pl.ds(sid * _STRIPE + q * zrows, zrows)])
        plsc.subcore_barrier()

        pltpu.async_copy(srcs_hbm.at[wid, 0], idx_s.at[0], isem)
        pltpu.async_copy(dsts_hbm.at[wid, 0], idx_d.at[0], isem)

        def chunk(c, p, u, first, last):
            pltpu.make_async_copy(srcs_hbm.at[wid, c], idx_s.at[p], isem).wait()
            pltpu.make_async_copy(dsts_hbm.at[wid, c], idx_d.at[p], isem).wait()
            ga = []
            for j in range(_KB):
                if first is None:
                    pltpu.make_async_copy(
                        rows.at[j], acc.at[idx_d.at[1 - p].at[j]], ssem
                    ).wait()
                else:
                    @pl.when(first)
                    def _(j=j):
                        pltpu.make_async_copy(
                            rows.at[j], acc.at[idx_d.at[1 - p].at[j]], ssem
                        ).wait()
                ga.append(
                    pltpu.async_copy(tbl_hbm.at[idx_s.at[p].at[j]], rows.at[j], gsem)
                )

            @pl.when(last)
            def _():
                pltpu.async_copy(srcs_hbm.at[wid, c + 1], idx_s.at[1 - p], isem)
                pltpu.async_copy(dsts_hbm.at[wid, c + 1], idx_d.at[1 - p], isem)

            for j in range(_KB):
                ga[j].wait()
                pltpu.async_copy(rows.at[j], acc.at[idx_d.at[p].at[j]], ssem, add=True)

        def body(u, carry):
            a = 2 * u
            chunk(a, 0, u, u > 0, a + 1 < T)
            chunk(a + 1, 1, u, None, a + 2 < T)
            return carry

        lax.fori_loop(0, T // 2, body, 0)
        for j in range(_KB):
            pltpu.make_async_copy(rows.at[j], acc.at[idx_d.at[1].at[j]], ssem).wait()
        plsc.subcore_barrier()
        for q in range(16):
            off = sid * _STRIPE + q * zrows
            pltpu.sync_copy(acc.at[pl.ds(off, zrows)], zbuf)
            pltpu.sync_copy(zbuf, out_hbm.at[cid, pl.ds(off, zrows)])

    return k(srcs, dsts, table, zeros_tbl)


def _dense_prep(deg_p, x_pad):
    def body(dp_ref, x_ref, dis_ref, g1_ref):
        deg = dp_ref[0, :, 0:1] + dp_ref[1, :, 0:1] + 1.0
        dis = lax.rsqrt(deg)
        dis_ref[...] = dis
        g1_ref[...] = x_ref[...] * dis

    return pl.pallas_call(
        body,
        grid=(_NPAD // _R,),
        in_specs=[
            pl.BlockSpec((2, _R, _DW), lambda i: (0, i, 0)),
            pl.BlockSpec((_R, _W), lambda i: (i, 0)),
        ],
        out_specs=[
            pl.BlockSpec((_R, 1), lambda i: (i, 0)),
            pl.BlockSpec((_R, _W), lambda i: (i, 0)),
        ],
        out_shape=[
            jax.ShapeDtypeStruct((_NPAD, 1), jnp.float32),
            jax.ShapeDtypeStruct((_NPAD, _W), jnp.float32),
        ],
    )(deg_p, x_pad)


def _dense_layer1(tp1, g1, dis, w_pad, b_pad):
    def body(tp_ref, g_ref, d_ref, w_ref, b_ref, ga_ref, gb_ref):
        s = tp_ref[0] + tp_ref[1] + g_ref[...]
        pre = s * d_ref[...]
        h = jnp.maximum(
            jnp.dot(pre, w_ref[...], preferred_element_type=jnp.float32)
            + b_ref[...],
            0.0,
        )
        ga_ref[...] = h[:, :_W] * d_ref[...]
        gb_ref[...] = h[:, _W:] * d_ref[...]

    return pl.pallas_call(
        body,
        grid=(_NPAD // _R,),
        in_specs=[
            pl.BlockSpec((2, _R, _W), lambda i: (0, i, 0)),
            pl.BlockSpec((_R, _W), lambda i: (i, 0)),
            pl.BlockSpec((_R, 1), lambda i: (i, 0)),
            pl.BlockSpec((_W, 2 * _W), lambda i: (0, 0)),
            pl.BlockSpec((1, 2 * _W), lambda i: (0, 0)),
        ],
        out_specs=[
            pl.BlockSpec((_R, _W), lambda i: (i, 0)),
            pl.BlockSpec((_R, _W), lambda i: (i, 0)),
        ],
        out_shape=[
            jax.ShapeDtypeStruct((_NPAD, _W), jnp.float32),
            jax.ShapeDtypeStruct((_NPAD, _W), jnp.float32),
        ],
    )(tp1, g1, dis, w_pad, b_pad)


def _dense_layer2(tpa, tpb, ga, gb, dis, w_pad, b_pad):
    def body(tpa_ref, tpb_ref, ga_ref, gb_ref, d_ref, w_ref, b_ref, oa_ref, ob_ref):
        sa = tpa_ref[0] + tpa_ref[1] + ga_ref[...]
        sb = tpb_ref[0] + tpb_ref[1] + gb_ref[...]
        pre = jnp.concatenate([sa, sb], axis=1) * d_ref[...]
        h = jnp.maximum(
            jnp.dot(pre, w_ref[...], preferred_element_type=jnp.float32)
            + b_ref[...],
            0.0,
        )
        oa_ref[...] = h[:, :_W] * d_ref[...]
        ob_ref[...] = h[:, _W:] * d_ref[...]

    return pl.pallas_call(
        body,
        grid=(_NPAD // _R,),
        in_specs=[
            pl.BlockSpec((2, _R, _W), lambda i: (0, i, 0)),
            pl.BlockSpec((2, _R, _W), lambda i: (0, i, 0)),
            pl.BlockSpec((_R, _W), lambda i: (i, 0)),
            pl.BlockSpec((_R, _W), lambda i: (i, 0)),
            pl.BlockSpec((_R, 1), lambda i: (i, 0)),
            pl.BlockSpec((2 * _W, 2 * _W), lambda i: (0, 0)),
            pl.BlockSpec((1, 2 * _W), lambda i: (0, 0)),
        ],
        out_specs=[
            pl.BlockSpec((_R, _W), lambda i: (i, 0)),
            pl.BlockSpec((_R, _W), lambda i: (i, 0)),
        ],
        out_shape=[
            jax.ShapeDtypeStruct((_NPAD, _W), jnp.float32),
            jax.ShapeDtypeStruct((_NPAD, _W), jnp.float32),
        ],
    )(tpa, tpb, ga, gb, dis, w_pad, b_pad)


def _dense_layer3(tpa, tpb, ga, gb, dis, w_pad, b_pad):
    def body(tpa_ref, tpb_ref, ga_ref, gb_ref, d_ref, w_ref, b_ref, o_ref):
        sa = tpa_ref[0] + tpa_ref[1] + ga_ref[...]
        sb = tpb_ref[0] + tpb_ref[1] + gb_ref[...]
        pre = jnp.concatenate([sa, sb], axis=1) * d_ref[...]
        o_ref[...] = (
            jnp.dot(pre, w_ref[...], preferred_element_type=jnp.float32)
            + b_ref[...]
        )

    return pl.pallas_call(
        body,
        grid=(_NPAD // _R,),
        in_specs=[
            pl.BlockSpec((2, _R, _W), lambda i: (0, i, 0)),
            pl.BlockSpec((2, _R, _W), lambda i: (0, i, 0)),
            pl.BlockSpec((_R, _W), lambda i: (i, 0)),
            pl.BlockSpec((_R, _W), lambda i: (i, 0)),
            pl.BlockSpec((_R, 1), lambda i: (i, 0)),
            pl.BlockSpec((2 * _W, 40), lambda i: (0, 0)),
            pl.BlockSpec((1, 40), lambda i: (0, 0)),
        ],
        out_specs=pl.BlockSpec((_R, 40), lambda i: (i, 0)),
        out_shape=jax.ShapeDtypeStruct((_NPAD, 40), jnp.float32),
    )(tpa, tpb, ga, gb, dis, w_pad, b_pad)


def kernel(x, edge_index, W1, b1, W2, b2, W3, b3):
    ei = edge_index.astype(jnp.int32)
    E = ei.shape[1]
    per_iter = _NW * _CH
    T = 2 * (-(-E // (2 * per_iter)))
    EP = T * per_iter
    pad = EP - E
    src = jnp.concatenate([ei[0], jnp.full((pad,), _N, jnp.int32)])
    dst = jnp.concatenate([ei[1], jnp.full((pad,), _N, jnp.int32)])
    srcs = src.reshape(_NW, T, _KB, 128)
    dsts = dst.reshape(_NW, T, _KB, 128)

    zeros_tbl = jnp.zeros((_STRIPE // 16, _W), jnp.float32)
    zeros_col = jnp.zeros((_STRIPE // 8, _DW), jnp.float32)
    ones = jnp.ones((128, _DW), jnp.float32)
    x_pad = jnp.zeros((_NPAD, _W), jnp.float32).at[:_N, :10].set(x)

    w1p = jnp.zeros((_W, 2 * _W), jnp.float32).at[:10, :20].set(W1)
    b1p = jnp.zeros((1, 2 * _W), jnp.float32).at[0, :20].set(b1)
    w2p = jnp.zeros((2 * _W, 2 * _W), jnp.float32).at[:20, :30].set(W2)
    b2p = jnp.zeros((1, 2 * _W), jnp.float32).at[0, :30].set(b2)
    w3p = jnp.zeros((2 * _W, 40), jnp.float32).at[:30, :40].set(W3)
    b3p = jnp.zeros((1, 40), jnp.float32).at[0, :40].set(b3)

    deg_p = _sc_degree(dsts, ones, zeros_col)
    dis, g1 = _dense_prep(deg_p, x_pad)

    tp1 = _sc_pass(srcs, dsts, g1, zeros_tbl)
    g2a, g2b = _dense_layer1(tp1, g1, dis, w1p, b1p)

    tp2a = _sc_pass(srcs, dsts, g2a, zeros_tbl)
    tp2b = _sc_pass(srcs, dsts, g2b, zeros_tbl)
    g3a, g3b = _dense_layer2(tp2a, tp2b, g2a, g2b, dis, w2p, b2p)

    tp3a = _sc_pass(srcs, dsts, g3a, zeros_tbl)
    tp3b = _sc_pass(srcs, dsts, g3b, zeros_tbl)
    out = _dense_layer3(tp3a, tp3b, g3a, g3b, dis, w3p, b3p)
    return out[:_N]


# speedup vs baseline: 1.1988x; 1.1988x over previous
import functools

import jax
import jax.numpy as jnp
from jax import lax
from jax.experimental import pallas as pl
from jax.experimental.pallas import tpu as pltpu
from jax.experimental.pallas import tpu_sc as plsc

_N = 100000
_NPAD = 100096
_W = 16
_NC = 2
_NS = 16
_NW = _NC * _NS
_KB = 8
_CH = _KB * 128
_STRIPE = _NPAD // _NS
_R = 2176


def _mesh():
    return plsc.VectorSubcoreMesh(
        core_axis_name="c", subcore_axis_name="s", num_cores=_NC, num_subcores=_NS
    )


_SC_PARAMS = pltpu.CompilerParams(use_tc_tiling_on_sc=False)


_DW = 8


def _sc_degree(dsts, ones, zeros_col):
    T = dsts.shape[1]

    @functools.partial(
        pl.kernel,
        out_type=jax.ShapeDtypeStruct((_NC, _NPAD, _DW), jnp.float32),
        mesh=_mesh(),
        scratch_types=[
            pltpu.VMEM((2, _KB, 128), jnp.int32),
            pltpu.VMEM((128, _DW), jnp.float32),
            pltpu.VMEM((_STRIPE // 8, _DW), jnp.float32),
            pltpu.VMEM_SHARED((_NPAD, _DW), jnp.float32),
            pltpu.SemaphoreType.DMA,
            pltpu.SemaphoreType.DMA,
        ],
        compiler_params=_SC_PARAMS,
    )
    def k(dsts_hbm, ones_hbm, z_hbm, out_hbm, idx_d, ones_v, zbuf, acc, isem, ssem):
        cid = lax.axis_index("c")
        sid = lax.axis_index("s")
        wid = cid * _NS + sid
        pltpu.sync_copy(ones_hbm, ones_v)
        pltpu.sync_copy(z_hbm, zbuf)
        for q in range(8):
            pltpu.sync_copy(
                zbuf, acc.at[pl.ds(sid * _STRIPE + q * (_STRIPE // 8), _STRIPE // 8)]
            )
        plsc.subcore_barrier()
        pltpu.async_copy(dsts_hbm.at[wid, 0], idx_d.at[0], isem)

        def chunk(c, p, first, last):
            pltpu.make_async_copy(dsts_hbm.at[wid, c], idx_d.at[p], isem).wait()
            for j in range(_KB):
                if first is None:
                    pltpu.make_async_copy(
                        ones_v, acc.at[idx_d.at[1 - p].at[j]], ssem
                    ).wait()
                else:
                    @pl.when(first)
                    def _(j=j):
                        pltpu.make_async_copy(
                            ones_v, acc.at[idx_d.at[1 - p].at[j]], ssem
                        ).wait()

            @pl.when(last)
            def _():
                pltpu.async_copy(dsts_hbm.at[wid, c + 1], idx_d.at[1 - p], isem)

            for j in range(_KB):
                pltpu.async_copy(ones_v, acc.at[idx_d.at[p].at[j]], ssem, add=True)

        def body(u, carry):
            a = 2 * u
            chunk(a, 0, u > 0, a + 1 < T)
            chunk(a + 1, 1, None, a + 2 < T)
            return carry

        lax.fori_loop(0, T // 2, body, 0)
        for j in range(_KB):
            pltpu.make_async_copy(ones_v, acc.at[idx_d.at[1].at[j]], ssem).wait()
        plsc.subcore_barrier()
        for q in range(8):
            off = sid * _STRIPE + q * (_STRIPE // 8)
            pltpu.sync_copy(acc.at[pl.ds(off, _STRIPE // 8)], zbuf)
            pltpu.sync_copy(zbuf, out_hbm.at[cid, pl.ds(off, _STRIPE // 8)])

    return k(dsts, ones, zeros_col)


def _sc_pass(srcs, dsts, table, zeros_tbl):
    T = srcs.shape[1]

    @functools.partial(
        pl.kernel,
        out_type=jax.ShapeDtypeStruct((_NC, _NPAD, _W), jnp.float32),
        mesh=_mesh(),
        scratch_types=[
            pltpu.VMEM((2, _KB, 128), jnp.int32),
            pltpu.VMEM((2, _KB, 128), jnp.int32),
            pltpu.VMEM((_KB, 128, _W), jnp.float32),
            pltpu.VMEM((_STRIPE // 16, _W), jnp.float32),
            pltpu.VMEM_SHARED((_NPAD, _W), jnp.float32),
            pltpu.SemaphoreType.DMA,
            pltpu.SemaphoreType.DMA,
            pltpu.SemaphoreType.DMA,
        ],
        compiler_params=_SC_PARAMS,
    )
    def k(srcs_hbm, dsts_hbm, tbl_hbm, z_hbm, out_hbm,
          idx_s, idx_d, rows, zbuf, acc, isem, gsem, ssem):
        cid = lax.axis_index("c")
        sid = lax.axis_index("s")
        wid = cid * _NS + sid
        zrows = _STRIPE // 16
        pltpu.sync_copy(z_hbm, zbuf)
        for q in range(16):
            pltpu.sync_copy(zbuf, acc.at[pl.ds(sid * _STRIPE + q * zrows, zrows)])
        plsc.subcore_barrier()

        pltpu.async_copy(srcs_hbm.at[wid, 0], idx_s.at[0], isem)
        pltpu.async_copy(dsts_hbm.at[wid, 0], idx_d.at[0], isem)

        def chunk(c, p, u, first, last):
            pltpu.make_async_copy(srcs_hbm.at[wid, c], idx_s.at[p], isem).wait()
            pltpu.make_async_copy(dsts_hbm.at[wid, c], idx_d.at[p], isem).wait()
            ga = []
            for j in range(_KB):
                if first is None:
                    pltpu.make_async_copy(
                        rows.at[j], acc.at[idx_d.at[1 - p].at[j]], ssem
                    ).wait()
                else:
                    @pl.when(first)
                    def _(j=j):
                        pltpu.make_async_copy(
                            rows.at[j], acc.at[idx_d.at[1 - p].at[j]], ssem
                        ).wait()
                ga.append(
                    pltpu.async_copy(tbl_hbm.at[idx_s.at[p].at[j]], rows.at[j], gsem)
                )

            @pl.when(last)
            def _():
                pltpu.async_copy(srcs_hbm.at[wid, c + 1], idx_s.at[1 - p], isem)
                pltpu.async_copy(dsts_hbm.at[wid, c + 1], idx_d.at[1 - p], isem)

            for j in range(_KB):
                ga[j].wait()
                pltpu.async_copy(rows.at[j], acc.at[idx_d.at[p].at[j]], ssem, add=True)

        def body(u, carry):
            a = 2 * u
            chunk(a, 0, u, u > 0, a + 1 < T)
            chunk(a + 1, 1, u, None, a + 2 < T)
            return carry

        lax.fori_loop(0, T // 2, body, 0)
        for j in range(_KB):
            pltpu.make_async_copy(rows.at[j], acc.at[idx_d.at[1].at[j]], ssem).wait()
        plsc.subcore_barrier()
        for q in range(16):
            off = sid * _STRIPE + q * zrows
            pltpu.sync_copy(acc.at[pl.ds(off, zrows)], zbuf)
            pltpu.sync_copy(zbuf, out_hbm.at[cid, pl.ds(off, zrows)])

    return k(srcs, dsts, table, zeros_tbl)


def _dense_prep(deg_p, x_pad):
    def body(dp_ref, x_ref, dis_ref, g1_ref):
        deg = dp_ref[0, :, 0:1] + dp_ref[1, :, 0:1] + 1.0
        dis = lax.rsqrt(deg)
        dis_ref[...] = dis
        g1_ref[...] = x_ref[...] * dis

    return pl.pallas_call(
        body,
        grid=(_NPAD // _R,),
        in_specs=[
            pl.BlockSpec((2, _R, _DW), lambda i: (0, i, 0)),
            pl.BlockSpec((_R, _W), lambda i: (i, 0)),
        ],
        out_specs=[
            pl.BlockSpec((_R, 1), lambda i: (i, 0)),
            pl.BlockSpec((_R, _W), lambda i: (i, 0)),
        ],
        out_shape=[
            jax.ShapeDtypeStruct((_NPAD, 1), jnp.float32),
            jax.ShapeDtypeStruct((_NPAD, _W), jnp.float32),
        ],
    )(deg_p, x_pad)


def _dense_layer1(tp1, g1, dis, w_pad, b_pad):
    def body(tp_ref, g_ref, d_ref, w_ref, b_ref, ga_ref, gb_ref):
        s = tp_ref[0] + tp_ref[1] + g_ref[...]
        pre = s * d_ref[...]
        h = jnp.maximum(
            jnp.dot(pre, w_ref[...], preferred_element_type=jnp.float32)
            + b_ref[...],
            0.0,
        )
        ga_ref[...] = h[:, :_W] * d_ref[...]
        gb_ref[...] = h[:, _W:] * d_ref[...]

    return pl.pallas_call(
        body,
        grid=(_NPAD // _R,),
        in_specs=[
            pl.BlockSpec((2, _R, _W), lambda i: (0, i, 0)),
            pl.BlockSpec((_R, _W), lambda i: (i, 0)),
            pl.BlockSpec((_R, 1), lambda i: (i, 0)),
            pl.BlockSpec((_W, 2 * _W), lambda i: (0, 0)),
            pl.BlockSpec((1, 2 * _W), lambda i: (0, 0)),
        ],
        out_specs=[
            pl.BlockSpec((_R, _W), lambda i: (i, 0)),
            pl.BlockSpec((_R, _W), lambda i: (i, 0)),
        ],
        out_shape=[
            jax.ShapeDtypeStruct((_NPAD, _W), jnp.float32),
            jax.ShapeDtypeStruct((_NPAD, _W), jnp.float32),
        ],
    )(tp1, g1, dis, w_pad, b_pad)


def _dense_layer2(tpa, tpb, ga, gb, dis, w_pad, b_pad):
    def body(tpa_ref, tpb_ref, ga_ref, gb_ref, d_ref, w_ref, b_ref, oa_ref, ob_ref):
        sa = tpa_ref[0] + tpa_ref[1] + ga_ref[...]
        sb = tpb_ref[0] + tpb_ref[1] + gb_ref[...]
        pre = jnp.concatenate([sa, sb], axis=1) * d_ref[...]
        h = jnp.maximum(
            jnp.dot(pre, w_ref[...], preferred_element_type=jnp.float32)
            + b_ref[...],
            0.0,
        )
        oa_ref[...] = h[:, :_W] * d_ref[...]
        ob_ref[...] = h[:, _W:] * d_ref[...]

    return pl.pallas_call(
        body,
        grid=(_NPAD // _R,),
        in_specs=[
            pl.BlockSpec((2, _R, _W), lambda i: (0, i, 0)),
            pl.BlockSpec((2, _R, _W), lambda i: (0, i, 0)),
            pl.BlockSpec((_R, _W), lambda i: (i, 0)),
            pl.BlockSpec((_R, _W), lambda i: (i, 0)),
            pl.BlockSpec((_R, 1), lambda i: (i, 0)),
            pl.BlockSpec((2 * _W, 2 * _W), lambda i: (0, 0)),
            pl.BlockSpec((1, 2 * _W), lambda i: (0, 0)),
        ],
        out_specs=[
            pl.BlockSpec((_R, _W), lambda i: (i, 0)),
            pl.BlockSpec((_R, _W), lambda i: (i, 0)),
        ],
        out_shape=[
            jax.ShapeDtypeStruct((_NPAD, _W), jnp.float32),
            jax.ShapeDtypeStruct((_NPAD, _W), jnp.float32),
        ],
    )(tpa, tpb, ga, gb, dis, w_pad, b_pad)


def _dense_layer3(tpa, tpb, ga, gb, dis, w_pad, b_pad):
    def body(tpa_ref, tpb_ref, ga_ref, gb_ref, d_ref, w_ref, b_ref, o_ref):
        sa = tpa_ref[0] + tpa_ref[1] + ga_ref[...]
        sb = tpb_ref[0] + tpb_ref[1] + gb_ref[...]
        pre = jnp.concatenate([sa, sb], axis=1) * d_ref[...]
        o_ref[...] = (
            jnp.dot(pre, w_ref[...], preferred_element_type=jnp.float32)
            + b_ref[...]
        )

    return pl.pallas_call(
        body,
        grid=(_NPAD // _R,),
        in_specs=[
            pl.BlockSpec((2, _R, _W), lambda i: (0, i, 0)),
            pl.BlockSpec((2, _R, _W), lambda i: (0, i, 0)),
            pl.BlockSpec((_R, _W), lambda i: (i, 0)),
            pl.BlockSpec((_R, _W), lambda i: (i, 0)),
            pl.BlockSpec((_R, 1), lambda i: (i, 0)),
            pl.BlockSpec((2 * _W, 40), lambda i: (0, 0)),
            pl.BlockSpec((1, 40), lambda i: (0, 0)),
        ],
        out_specs=pl.BlockSpec((_R, 40), lambda i: (i, 0)),
        out_shape=jax.ShapeDtypeStruct((_NPAD, 40), jnp.float32),
    )(tpa, tpb, ga, gb, dis, w_pad, b_pad)


def kernel(x, edge_index, W1, b1, W2, b2, W3, b3):
    ei = edge_index.astype(jnp.int32)
    E = ei.shape[1]
    per_iter = _NW * _CH
    T = 2 * (-(-E // (2 * per_iter)))
    EP = T * per_iter
    pad = EP - E
    src = jnp.concatenate([ei[0], jnp.full((pad,), _N, jnp.int32)])
    dst = jnp.concatenate([ei[1], jnp.full((pad,), _N, jnp.int32)])
    srcs = src.reshape(_NW, T, _KB, 128)
    dsts = dst.reshape(_NW, T, _KB, 128)

    zeros_tbl = jnp.zeros((_STRIPE // 16, _W), jnp.float32)
    zeros_col = jnp.zeros((_STRIPE // 8, _DW), jnp.float32)
    ones = jnp.ones((128, _DW), jnp.float32)
    x_pad = jnp.zeros((_NPAD, _W), jnp.float32).at[:_N, :10].set(x)

    w1p = jnp.zeros((_W, 2 * _W), jnp.float32).at[:10, :20].set(W1)
    b1p = jnp.zeros((1, 2 * _W), jnp.float32).at[0, :20].set(b1)
    w2p = jnp.zeros((2 * _W, 2 * _W), jnp.float32).at[:20, :30].set(W2)
    b2p = jnp.zeros((1, 2 * _W), jnp.float32).at[0, :30].set(b2)
    w3p = jnp.zeros((2 * _W, 40), jnp.float32).at[:30, :40].set(W3)
    b3p = jnp.zeros((1, 40), jnp.float32).at[0, :40].set(b3)

    deg_p = _sc_degree(dsts, ones, zeros_col)
    dis, g1 = _dense_prep(deg_p, x_pad)

    tp1 = _sc_pass(srcs, dsts, g1, zeros_tbl)
    g2a, g2b = _dense_layer1(tp1, g1, dis, w1p, b1p)

    tp2a = _sc_pass(srcs, dsts, g2a, zeros_tbl)
    tp2b = _sc_pass(srcs, dsts, g2b, zeros_tbl)
    g3a, g3b = _dense_layer2(tp2a, tp2b, g2a, g2b, dis, w2p, b2p)

    tp3a = _sc_pass(srcs, dsts, g3a, zeros_tbl)
    tp3b = _sc_pass(srcs, dsts, g3b, zeros_tbl)
    out = _dense_layer3(tp3a, tp3b, g3a, g3b, dis, w3p, b3p)
    return out[:_N]


# speedup vs baseline: 1.2653x; 1.0554x over previous
import functools

import jax
import jax.numpy as jnp
from jax import lax
from jax.experimental import pallas as pl
from jax.experimental.pallas import tpu as pltpu
from jax.experimental.pallas import tpu_sc as plsc

_N = 100000
_NPAD = 100096
_W = 16
_NC = 2
_NS = 16
_NW = _NC * _NS
_KB = 8
_CH = _KB * 128
_STRIPE = _NPAD // _NS
_R = 2176


def _mesh():
    return plsc.VectorSubcoreMesh(
        core_axis_name="c", subcore_axis_name="s", num_cores=_NC, num_subcores=_NS
    )


_SC_PARAMS = pltpu.CompilerParams(use_tc_tiling_on_sc=False)


_DW = 8


def _sc_degree(dsts, ones, zeros_col, T0, T1):

    @functools.partial(
        pl.kernel,
        out_type=jax.ShapeDtypeStruct((_NC, _NPAD, _DW), jnp.float32),
        mesh=_mesh(),
        scratch_types=[
            pltpu.VMEM((2, _KB, 128), jnp.int32),
            pltpu.VMEM((128, _DW), jnp.float32),
            pltpu.VMEM((_STRIPE // 8, _DW), jnp.float32),
            pltpu.VMEM_SHARED((_NPAD, _DW), jnp.float32),
            pltpu.SemaphoreType.DMA,
            pltpu.SemaphoreType.DMA,
        ],
        compiler_params=_SC_PARAMS,
    )
    def k(dsts_hbm, ones_hbm, z_hbm, out_hbm, idx_d, ones_v, zbuf, acc, isem, ssem):
        cid = lax.axis_index("c")
        sid = lax.axis_index("s")
        base = cid * (_NS * T0) + sid * jnp.where(cid == 0, T0, T1)
        Tw = jnp.where(cid == 0, T0, T1)
        pltpu.sync_copy(ones_hbm, ones_v)
        pltpu.sync_copy(z_hbm, zbuf)
        for q in range(8):
            pltpu.sync_copy(
                zbuf, acc.at[pl.ds(sid * _STRIPE + q * (_STRIPE // 8), _STRIPE // 8)]
            )
        plsc.subcore_barrier()
        pltpu.async_copy(dsts_hbm.at[base], idx_d.at[0], isem)

        def chunk(c, p, first, last):
            pltpu.make_async_copy(dsts_hbm.at[base + c], idx_d.at[p], isem).wait()
            for j in range(_KB):
                if first is None:
                    pltpu.make_async_copy(
                        ones_v, acc.at[idx_d.at[1 - p].at[j]], ssem
                    ).wait()
                else:
                    @pl.when(first)
                    def _(j=j):
                        pltpu.make_async_copy(
                            ones_v, acc.at[idx_d.at[1 - p].at[j]], ssem
                        ).wait()

            @pl.when(last)
            def _():
                pltpu.async_copy(dsts_hbm.at[base + c + 1], idx_d.at[1 - p], isem)

            for j in range(_KB):
                pltpu.async_copy(ones_v, acc.at[idx_d.at[p].at[j]], ssem, add=True)

        def body(u, carry):
            a = 2 * u
            chunk(a, 0, u > 0, a + 1 < Tw)
            chunk(a + 1, 1, None, a + 2 < Tw)
            return carry

        lax.fori_loop(0, Tw // 2, body, 0)
        for j in range(_KB):
            pltpu.make_async_copy(ones_v, acc.at[idx_d.at[1].at[j]], ssem).wait()
        plsc.subcore_barrier()
        for q in range(8):
            off = sid * _STRIPE + q * (_STRIPE // 8)
            pltpu.sync_copy(acc.at[pl.ds(off, _STRIPE // 8)], zbuf)
            pltpu.sync_copy(zbuf, out_hbm.at[cid, pl.ds(off, _STRIPE // 8)])

    return k(dsts, ones, zeros_col)


def _sc_pass(srcs, dsts, table, zeros_tbl, T0, T1):

    @functools.partial(
        pl.kernel,
        out_type=jax.ShapeDtypeStruct((_NC, _NPAD, _W), jnp.float32),
        mesh=_mesh(),
        scratch_types=[
            pltpu.VMEM((2, _KB, 128), jnp.int32),
            pltpu.VMEM((2, _KB, 128), jnp.int32),
            pltpu.VMEM((_KB, 128, _W), jnp.float32),
            pltpu.VMEM((_STRIPE // 16, _W), jnp.float32),
            pltpu.VMEM_SHARED((_NPAD, _W), jnp.float32),
            pltpu.SemaphoreType.DMA,
            pltpu.SemaphoreType.DMA,
            pltpu.SemaphoreType.DMA,
        ],
        compiler_params=_SC_PARAMS,
    )
    def k(srcs_hbm, dsts_hbm, tbl_hbm, z_hbm, out_hbm,
          idx_s, idx_d, rows, zbuf, acc, isem, gsem, ssem):
        cid = lax.axis_index("c")
        sid = lax.axis_index("s")
        base = cid * (_NS * T0) + sid * jnp.where(cid == 0, T0, T1)
        Tw = jnp.where(cid == 0, T0, T1)
        zrows = _STRIPE // 16
        pltpu.sync_copy(z_hbm, zbuf)
        for q in range(16):
            pltpu.sync_copy(zbuf, acc.at[pl.ds(sid * _STRIPE + q * zrows, zrows)])
        plsc.subcore_barrier()

        pltpu.async_copy(srcs_hbm.at[base], idx_s.at[0], isem)
        pltpu.async_copy(dsts_hbm.at[base], idx_d.at[0], isem)

        def chunk(c, p, u, first, last):
            pltpu.make_async_copy(srcs_hbm.at[base + c], idx_s.at[p], isem).wait()
            pltpu.make_async_copy(dsts_hbm.at[base + c], idx_d.at[p], isem).wait()
            ga = []
            for j in range(_KB):
                if first is None:
                    pltpu.make_async_copy(
                        rows.at[j], acc.at[idx_d.at[1 - p].at[j]], ssem
                    ).wait()
                else:
                    @pl.when(first)
                    def _(j=j):
                        pltpu.make_async_copy(
                            rows.at[j], acc.at[idx_d.at[1 - p].at[j]], ssem
                        ).wait()
                ga.append(
                    pltpu.async_copy(tbl_hbm.at[idx_s.at[p].at[j]], rows.at[j], gsem)
                )

            @pl.when(last)
            def _():
                pltpu.async_copy(srcs_hbm.at[base + c + 1], idx_s.at[1 - p], isem)
                pltpu.async_copy(dsts_hbm.at[base + c + 1], idx_d.at[1 - p], isem)

            for j in range(_KB):
                ga[j].wait()
                pltpu.async_copy(rows.at[j], acc.at[idx_d.at[p].at[j]], ssem, add=True)

        def body(u, carry):
            a = 2 * u
            chunk(a, 0, u, u > 0, a + 1 < Tw)
            chunk(a + 1, 1, u, None, a + 2 < Tw)
            return carry

        lax.fori_loop(0, Tw // 2, body, 0)
        for j in range(_KB):
            pltpu.make_async_copy(rows.at[j], acc.at[idx_d.at[1].at[j]], ssem).wait()
        plsc.subcore_barrier()
        for q in range(16):
            off = sid * _STRIPE + q * zrows
            pltpu.sync_copy(acc.at[pl.ds(off, zrows)], zbuf)
            pltpu.sync_copy(zbuf, out_hbm.at[cid, pl.ds(off, zrows)])

    return k(srcs, dsts, table, zeros_tbl)


def _dense_prep(deg_p, x_pad):
    def body(dp_ref, x_ref, dis_ref, g1_ref):
        deg = dp_ref[0, :, 0:1] + dp_ref[1, :, 0:1] + 1.0
        dis = lax.rsqrt(deg)
        dis_ref[...] = dis
        g1_ref[...] = x_ref[...] * dis

    return pl.pallas_call(
        body,
        grid=(_NPAD // _R,),
        in_specs=[
            pl.BlockSpec((2, _R, _DW), lambda i: (0, i, 0)),
            pl.BlockSpec((_R, _W), lambda i: (i, 0)),
        ],
        out_specs=[
            pl.BlockSpec((_R, 1), lambda i: (i, 0)),
            pl.BlockSpec((_R, _W), lambda i: (i, 0)),
        ],
        out_shape=[
            jax.ShapeDtypeStruct((_NPAD, 1), jnp.float32),
            jax.ShapeDtypeStruct((_NPAD, _W), jnp.float32),
        ],
    )(deg_p, x_pad)


def _dense_layer1(tp1, g1, dis, w_pad, b_pad):
    def body(tp_ref, g_ref, d_ref, w_ref, b_ref, ga_ref, gb_ref):
        s = tp_ref[0] + tp_ref[1] + g_ref[...]
        pre = s * d_ref[...]
        h = jnp.maximum(
            jnp.dot(pre, w_ref[...], preferred_element_type=jnp.float32)
            + b_ref[...],
            0.0,
        )
        ga_ref[...] = h[:, :_W] * d_ref[...]
        gb_ref[...] = h[:, _W:] * d_ref[...]

    return pl.pallas_call(
        body,
        grid=(_NPAD // _R,),
        in_specs=[
            pl.BlockSpec((2, _R, _W), lambda i: (0, i, 0)),
            pl.BlockSpec((_R, _W), lambda i: (i, 0)),
            pl.BlockSpec((_R, 1), lambda i: (i, 0)),
            pl.BlockSpec((_W, 2 * _W), lambda i: (0, 0)),
            pl.BlockSpec((1, 2 * _W), lambda i: (0, 0)),
        ],
        out_specs=[
            pl.BlockSpec((_R, _W), lambda i: (i, 0)),
            pl.BlockSpec((_R, _W), lambda i: (i, 0)),
        ],
        out_shape=[
            jax.ShapeDtypeStruct((_NPAD, _W), jnp.float32),
            jax.ShapeDtypeStruct((_NPAD, _W), jnp.float32),
        ],
    )(tp1, g1, dis, w_pad, b_pad)


def _dense_layer2(tpa, tpb, ga, gb, dis, w_pad, b_pad):
    def body(tpa_ref, tpb_ref, ga_ref, gb_ref, d_ref, w_ref, b_ref, oa_ref, ob_ref):
        sa = tpa_ref[0] + tpa_ref[1] + ga_ref[...]
        sb = tpb_ref[0] + tpb_ref[1] + gb_ref[...]
        pre = jnp.concatenate([sa, sb], axis=1) * d_ref[...]
        h = jnp.maximum(
            jnp.dot(pre, w_ref[...], preferred_element_type=jnp.float32)
            + b_ref[...],
            0.0,
        )
        oa_ref[...] = h[:, :_W] * d_ref[...]
        ob_ref[...] = h[:, _W:] * d_ref[...]

    return pl.pallas_call(
        body,
        grid=(_NPAD // _R,),
        in_specs=[
            pl.BlockSpec((2, _R, _W), lambda i: (0, i, 0)),
            pl.BlockSpec((2, _R, _W), lambda i: (0, i, 0)),
            pl.BlockSpec((_R, _W), lambda i: (i, 0)),
            pl.BlockSpec((_R, _W), lambda i: (i, 0)),
            pl.BlockSpec((_R, 1), lambda i: (i, 0)),
            pl.BlockSpec((2 * _W, 2 * _W), lambda i: (0, 0)),
            pl.BlockSpec((1, 2 * _W), lambda i: (0, 0)),
        ],
        out_specs=[
            pl.BlockSpec((_R, _W), lambda i: (i, 0)),
            pl.BlockSpec((_R, _W), lambda i: (i, 0)),
        ],
        out_shape=[
            jax.ShapeDtypeStruct((_NPAD, _W), jnp.float32),
            jax.ShapeDtypeStruct((_NPAD, _W), jnp.float32),
        ],
    )(tpa, tpb, ga, gb, dis, w_pad, b_pad)


def _dense_layer3(tpa, tpb, ga, gb, dis, w_pad, b_pad):
    def body(tpa_ref, tpb_ref, ga_ref, gb_ref, d_ref, w_ref, b_ref, o_ref):
        sa = tpa_ref[0] + tpa_ref[1] + ga_ref[...]
        sb = tpb_ref[0] + tpb_ref[1] + gb_ref[...]
        pre = jnp.concatenate([sa, sb], axis=1) * d_ref[...]
        o_ref[...] = (
            jnp.dot(pre, w_ref[...], preferred_element_type=jnp.float32)
            + b_ref[...]
        )

    return pl.pallas_call(
        body,
        grid=(_NPAD // _R,),
        in_specs=[
            pl.BlockSpec((2, _R, _W), lambda i: (0, i, 0)),
            pl.BlockSpec((2, _R, _W), lambda i: (0, i, 0)),
            pl.BlockSpec((_R, _W), lambda i: (i, 0)),
            pl.BlockSpec((_R, _W), lambda i: (i, 0)),
            pl.BlockSpec((_R, 1), lambda i: (i, 0)),
            pl.BlockSpec((2 * _W, 40), lambda i: (0, 0)),
            pl.BlockSpec((1, 40), lambda i: (0, 0)),
        ],
        out_specs=pl.BlockSpec((_R, 40), lambda i: (i, 0)),
        out_shape=jax.ShapeDtypeStruct((_NPAD, 40), jnp.float32),
    )(tpa, tpb, ga, gb, dis, w_pad, b_pad)


def kernel(x, edge_index, W1, b1, W2, b2, W3, b3):
    ei = edge_index.astype(jnp.int32)
    E = ei.shape[1]
    S = 2 * (-(-E // (2 * _NS * _CH)))
    T0 = 2 * round(S * 0.566 / 2)
    T1 = S - T0
    TOT = _NS * (T0 + T1)
    EP = TOT * _CH
    pad = EP - E
    src = jnp.concatenate([ei[0], jnp.full((pad,), _N, jnp.int32)])
    dst = jnp.concatenate([ei[1], jnp.full((pad,), _N, jnp.int32)])
    srcs = src.reshape(TOT, _KB, 128)
    dsts = dst.reshape(TOT, _KB, 128)

    zeros_tbl = jnp.zeros((_STRIPE // 16, _W), jnp.float32)
    zeros_col = jnp.zeros((_STRIPE // 8, _DW), jnp.float32)
    ones = jnp.ones((128, _DW), jnp.float32)
    x_pad = jnp.zeros((_NPAD, _W), jnp.float32).at[:_N, :10].set(x)

    w1p = jnp.zeros((_W, 2 * _W), jnp.float32).at[:10, :20].set(W1)
    b1p = jnp.zeros((1, 2 * _W), jnp.float32).at[0, :20].set(b1)
    w2p = jnp.zeros((2 * _W, 2 * _W), jnp.float32).at[:20, :30].set(W2)
    b2p = jnp.zeros((1, 2 * _W), jnp.float32).at[0, :30].set(b2)
    w3p = jnp.zeros((2 * _W, 40), jnp.float32).at[:30, :40].set(W3)
    b3p = jnp.zeros((1, 40), jnp.float32).at[0, :40].set(b3)

    deg_p = _sc_degree(dsts, ones, zeros_col, T0, T1)
    dis, g1 = _dense_prep(deg_p, x_pad)

    tp1 = _sc_pass(srcs, dsts, g1, zeros_tbl, T0, T1)
    g2a, g2b = _dense_layer1(tp1, g1, dis, w1p, b1p)

    tp2a = _sc_pass(srcs, dsts, g2a, zeros_tbl, T0, T1)
    tp2b = _sc_pass(srcs, dsts, g2b, zeros_tbl, T0, T1)
    g3a, g3b = _dense_layer2(tp2a, tp2b, g2a, g2b, dis, w2p, b2p)

    tp3a = _sc_pass(srcs, dsts, g3a, zeros_tbl, T0, T1)
    tp3b = _sc_pass(srcs, dsts, g3b, zeros_tbl, T0, T1)
    out = _dense_layer3(tp3a, tp3b, g3a, g3b, dis, w3p, b3p)
    return out[:_N]


# speedup vs baseline: 1.2771x; 1.0094x over previous
import functools

import jax
import jax.numpy as jnp
from jax import lax
from jax.experimental import pallas as pl
from jax.experimental.pallas import tpu as pltpu
from jax.experimental.pallas import tpu_sc as plsc

_N = 100000
_NPAD = 100096
_W = 16
_NC = 2
_NS = 16
_NW = _NC * _NS
_KB = 8
_CH = _KB * 128
_STRIPE = _NPAD // _NS
_R = 2176


def _mesh():
    return plsc.VectorSubcoreMesh(
        core_axis_name="c", subcore_axis_name="s", num_cores=_NC, num_subcores=_NS
    )


_SC_PARAMS = pltpu.CompilerParams(use_tc_tiling_on_sc=False)


_DW = 8


def _sc_degree(dsts, ones, zeros_col, T0, T1):

    @functools.partial(
        pl.kernel,
        out_type=jax.ShapeDtypeStruct((_NC, _NPAD, _DW), jnp.float32),
        mesh=_mesh(),
        scratch_types=[
            pltpu.VMEM((2, _KB, 128), jnp.int32),
            pltpu.VMEM((128, _DW), jnp.float32),
            pltpu.VMEM((_STRIPE // 8, _DW), jnp.float32),
            pltpu.VMEM_SHARED((_NPAD, _DW), jnp.float32),
            pltpu.SemaphoreType.DMA,
            pltpu.SemaphoreType.DMA,
        ],
        compiler_params=_SC_PARAMS,
    )
    def k(dsts_hbm, ones_hbm, z_hbm, out_hbm, idx_d, ones_v, zbuf, acc, isem, ssem):
        cid = lax.axis_index("c")
        sid = lax.axis_index("s")
        base = cid * (_NS * T0) + sid * jnp.where(cid == 0, T0, T1)
        Tw = jnp.where(cid == 0, T0, T1)
        pltpu.sync_copy(ones_hbm, ones_v)
        pltpu.sync_copy(z_hbm, zbuf)
        for q in range(8):
            pltpu.sync_copy(
                zbuf, acc.at[pl.ds(sid * _STRIPE + q * (_STRIPE // 8), _STRIPE // 8)]
            )
        plsc.subcore_barrier()
        pltpu.async_copy(dsts_hbm.at[base], idx_d.at[0], isem)

        def chunk(c, p, first, last):
            pltpu.make_async_copy(dsts_hbm.at[base + c], idx_d.at[p], isem).wait()
            for j in range(_KB):
                if first is None:
                    pltpu.make_async_copy(
                        ones_v, acc.at[idx_d.at[1 - p].at[j]], ssem
                    ).wait()
                else:
                    @pl.when(first)
                    def _(j=j):
                        pltpu.make_async_copy(
                            ones_v, acc.at[idx_d.at[1 - p].at[j]], ssem
                        ).wait()

            @pl.when(last)
            def _():
                pltpu.async_copy(dsts_hbm.at[base + c + 1], idx_d.at[1 - p], isem)

            for j in range(_KB):
                pltpu.async_copy(ones_v, acc.at[idx_d.at[p].at[j]], ssem, add=True)

        def body(u, carry):
            a = 2 * u
            chunk(a, 0, u > 0, a + 1 < Tw)
            chunk(a + 1, 1, None, a + 2 < Tw)
            return carry

        lax.fori_loop(0, Tw // 2, body, 0)
        for j in range(_KB):
            pltpu.make_async_copy(ones_v, acc.at[idx_d.at[1].at[j]], ssem).wait()
        plsc.subcore_barrier()
        for q in range(8):
            off = sid * _STRIPE + q * (_STRIPE // 8)
            pltpu.sync_copy(acc.at[pl.ds(off, _STRIPE // 8)], zbuf)
            pltpu.sync_copy(zbuf, out_hbm.at[cid, pl.ds(off, _STRIPE // 8)])

    return k(dsts, ones, zeros_col)


def _sc_pass(srcs, dsts, table, zeros_tbl, T0, T1):

    @functools.partial(
        pl.kernel,
        out_type=jax.ShapeDtypeStruct((_NC, _NPAD, _W), jnp.float32),
        mesh=_mesh(),
        scratch_types=[
            pltpu.VMEM((2, _KB, 128), jnp.int32),
            pltpu.VMEM((2, _KB, 128), jnp.int32),
            pltpu.VMEM((_KB, 128, _W), jnp.float32),
            pltpu.VMEM((_STRIPE // 16, _W), jnp.float32),
            pltpu.VMEM_SHARED((_NPAD, _W), jnp.float32),
            pltpu.SemaphoreType.DMA,
            pltpu.SemaphoreType.DMA,
            pltpu.SemaphoreType.DMA,
        ],
        compiler_params=_SC_PARAMS,
    )
    def k(srcs_hbm, dsts_hbm, tbl_hbm, z_hbm, out_hbm,
          idx_s, idx_d, rows, zbuf, acc, isem, gsem, ssem):
        cid = lax.axis_index("c")
        sid = lax.axis_index("s")
        base = cid * (_NS * T0) + sid * jnp.where(cid == 0, T0, T1)
        Tw = jnp.where(cid == 0, T0, T1)
        zrows = _STRIPE // 16
        pltpu.sync_copy(z_hbm, zbuf)
        for q in range(16):
            pltpu.sync_copy(zbuf, acc.at[pl.ds(sid * _STRIPE + q * zrows, zrows)])
        plsc.subcore_barrier()

        pltpu.async_copy(srcs_hbm.at[base], idx_s.at[0], isem)
        pltpu.async_copy(dsts_hbm.at[base], idx_d.at[0], isem)

        def chunk(c, p, u, first, last):
            pltpu.make_async_copy(srcs_hbm.at[base + c], idx_s.at[p], isem).wait()
            pltpu.make_async_copy(dsts_hbm.at[base + c], idx_d.at[p], isem).wait()
            ga = []
            for j in range(_KB):
                if first is None:
                    pltpu.make_async_copy(
                        rows.at[j], acc.at[idx_d.at[1 - p].at[j]], ssem
                    ).wait()
                else:
                    @pl.when(first)
                    def _(j=j):
                        pltpu.make_async_copy(
                            rows.at[j], acc.at[idx_d.at[1 - p].at[j]], ssem
                        ).wait()
                ga.append(
                    pltpu.async_copy(tbl_hbm.at[idx_s.at[p].at[j]], rows.at[j], gsem)
                )

            @pl.when(last)
            def _():
                pltpu.async_copy(srcs_hbm.at[base + c + 1], idx_s.at[1 - p], isem)
                pltpu.async_copy(dsts_hbm.at[base + c + 1], idx_d.at[1 - p], isem)

            for j in range(_KB):
                ga[j].wait()
                pltpu.async_copy(rows.at[j], acc.at[idx_d.at[p].at[j]], ssem, add=True)

        def body(u, carry):
            a = 2 * u
            chunk(a, 0, u, u > 0, a + 1 < Tw)
            chunk(a + 1, 1, u, None, a + 2 < Tw)
            return carry

        lax.fori_loop(0, Tw // 2, body, 0)
        for j in range(_KB):
            pltpu.make_async_copy(rows.at[j], acc.at[idx_d.at[1].at[j]], ssem).wait()
        plsc.subcore_barrier()
        for q in range(16):
            off = sid * _STRIPE + q * zrows
            pltpu.sync_copy(acc.at[pl.ds(off, zrows)], zbuf)
            pltpu.sync_copy(zbuf, out_hbm.at[cid, pl.ds(off, zrows)])

    return k(srcs, dsts, table, zeros_tbl)


def _dense_prep(deg_p, x_pad):
    def body(dp_ref, x_ref, dis_ref, g1_ref):
        deg = dp_ref[0, :, 0:1] + dp_ref[1, :, 0:1] + 1.0
        dis = lax.rsqrt(deg)
        dis_ref[...] = dis
        g1_ref[...] = x_ref[...] * dis

    return pl.pallas_call(
        body,
        grid=(_NPAD // _R,),
        in_specs=[
            pl.BlockSpec((2, _R, _DW), lambda i: (0, i, 0)),
            pl.BlockSpec((_R, _W), lambda i: (i, 0)),
        ],
        out_specs=[
            pl.BlockSpec((_R, 1), lambda i: (i, 0)),
            pl.BlockSpec((_R, _W), lambda i: (i, 0)),
        ],
        out_shape=[
            jax.ShapeDtypeStruct((_NPAD, 1), jnp.float32),
            jax.ShapeDtypeStruct((_NPAD, _W), jnp.float32),
        ],
    )(deg_p, x_pad)


def _dense_layer1(tp1, g1, dis, w_pad, b_pad):
    def body(tp_ref, g_ref, d_ref, w_ref, b_ref, ga_ref, gb_ref):
        s = tp_ref[0] + tp_ref[1] + g_ref[...]
        pre = s * d_ref[...]
        h = jnp.maximum(
            jnp.dot(pre, w_ref[...], preferred_element_type=jnp.float32)
            + b_ref[...],
            0.0,
        )
        ga_ref[...] = h[:, :_W] * d_ref[...]
        gb_ref[...] = h[:, _W:] * d_ref[...]

    return pl.pallas_call(
        body,
        grid=(_NPAD // _R,),
        in_specs=[
            pl.BlockSpec((2, _R, _W), lambda i: (0, i, 0)),
            pl.BlockSpec((_R, _W), lambda i: (i, 0)),
            pl.BlockSpec((_R, 1), lambda i: (i, 0)),
            pl.BlockSpec((_W, 2 * _W), lambda i: (0, 0)),
            pl.BlockSpec((1, 2 * _W), lambda i: (0, 0)),
        ],
        out_specs=[
            pl.BlockSpec((_R, _W), lambda i: (i, 0)),
            pl.BlockSpec((_R, _W), lambda i: (i, 0)),
        ],
        out_shape=[
            jax.ShapeDtypeStruct((_NPAD, _W), jnp.float32),
            jax.ShapeDtypeStruct((_NPAD, _W), jnp.float32),
        ],
    )(tp1, g1, dis, w_pad, b_pad)


def _dense_layer2(tpa, tpb, ga, gb, dis, w_pad, b_pad):
    def body(tpa_ref, tpb_ref, ga_ref, gb_ref, d_ref, w_ref, b_ref, oa_ref, ob_ref):
        sa = tpa_ref[0] + tpa_ref[1] + ga_ref[...]
        sb = tpb_ref[0] + tpb_ref[1] + gb_ref[...]
        pre = jnp.concatenate([sa, sb], axis=1) * d_ref[...]
        h = jnp.maximum(
            jnp.dot(pre, w_ref[...], preferred_element_type=jnp.float32)
            + b_ref[...],
            0.0,
        )
        oa_ref[...] = h[:, :_W] * d_ref[...]
        ob_ref[...] = h[:, _W:] * d_ref[...]

    return pl.pallas_call(
        body,
        grid=(_NPAD // _R,),
        in_specs=[
            pl.BlockSpec((2, _R, _W), lambda i: (0, i, 0)),
            pl.BlockSpec((2, _R, _W), lambda i: (0, i, 0)),
            pl.BlockSpec((_R, _W), lambda i: (i, 0)),
            pl.BlockSpec((_R, _W), lambda i: (i, 0)),
            pl.BlockSpec((_R, 1), lambda i: (i, 0)),
            pl.BlockSpec((2 * _W, 2 * _W), lambda i: (0, 0)),
            pl.BlockSpec((1, 2 * _W), lambda i: (0, 0)),
        ],
        out_specs=[
            pl.BlockSpec((_R, _W), lambda i: (i, 0)),
            pl.BlockSpec((_R, _W), lambda i: (i, 0)),
        ],
        out_shape=[
            jax.ShapeDtypeStruct((_NPAD, _W), jnp.float32),
            jax.ShapeDtypeStruct((_NPAD, _W), jnp.float32),
        ],
    )(tpa, tpb, ga, gb, dis, w_pad, b_pad)


def _dense_layer3(tpa, tpb, ga, gb, dis, w_pad, b_pad):
    def body(tpa_ref, tpb_ref, ga_ref, gb_ref, d_ref, w_ref, b_ref, o_ref):
        sa = tpa_ref[0] + tpa_ref[1] + ga_ref[...]
        sb = tpb_ref[0] + tpb_ref[1] + gb_ref[...]
        pre = jnp.concatenate([sa, sb], axis=1) * d_ref[...]
        o_ref[...] = (
            jnp.dot(pre, w_ref[...], preferred_element_type=jnp.float32)
            + b_ref[...]
        )

    return pl.pallas_call(
        body,
        grid=(_NPAD // _R,),
        in_specs=[
            pl.BlockSpec((2, _R, _W), lambda i: (0, i, 0)),
            pl.BlockSpec((2, _R, _W), lambda i: (0, i, 0)),
            pl.BlockSpec((_R, _W), lambda i: (i, 0)),
            pl.BlockSpec((_R, _W), lambda i: (i, 0)),
            pl.BlockSpec((_R, 1), lambda i: (i, 0)),
            pl.BlockSpec((2 * _W, 40), lambda i: (0, 0)),
            pl.BlockSpec((1, 40), lambda i: (0, 0)),
        ],
        out_specs=pl.BlockSpec((_R, 40), lambda i: (i, 0)),
        out_shape=jax.ShapeDtypeStruct((_NPAD, 40), jnp.float32),
    )(tpa, tpb, ga, gb, dis, w_pad, b_pad)


def kernel(x, edge_index, W1, b1, W2, b2, W3, b3):
    ei = edge_index.astype(jnp.int32)
    E = ei.shape[1]
    S = 2 * (-(-E // (2 * _NS * _CH)))
    T0 = 2 * round(S * 0.589 / 2)
    T1 = S - T0
    TOT = _NS * (T0 + T1)
    EP = TOT * _CH
    pad = EP - E
    src = jnp.concatenate([ei[0], jnp.full((pad,), _N, jnp.int32)])
    dst = jnp.concatenate([ei[1], jnp.full((pad,), _N, jnp.int32)])
    srcs = src.reshape(TOT, _KB, 128)
    dsts = dst.reshape(TOT, _KB, 128)

    zeros_tbl = jnp.zeros((_STRIPE // 16, _W), jnp.float32)
    zeros_col = jnp.zeros((_STRIPE // 8, _DW), jnp.float32)
    ones = jnp.ones((128, _DW), jnp.float32)
    x_pad = jnp.zeros((_NPAD, _W), jnp.float32).at[:_N, :10].set(x)

    w1p = jnp.zeros((_W, 2 * _W), jnp.float32).at[:10, :20].set(W1)
    b1p = jnp.zeros((1, 2 * _W), jnp.float32).at[0, :20].set(b1)
    w2p = jnp.zeros((2 * _W, 2 * _W), jnp.float32).at[:20, :30].set(W2)
    b2p = jnp.zeros((1, 2 * _W), jnp.float32).at[0, :30].set(b2)
    w3p = jnp.zeros((2 * _W, 40), jnp.float32).at[:30, :40].set(W3)
    b3p = jnp.zeros((1, 40), jnp.float32).at[0, :40].set(b3)

    deg_p = _sc_degree(dsts, ones, zeros_col, T0, T1)
    dis, g1 = _dense_prep(deg_p, x_pad)

    tp1 = _sc_pass(srcs, dsts, g1, zeros_tbl, T0, T1)
    g2a, g2b = _dense_layer1(tp1, g1, dis, w1p, b1p)

    tp2a = _sc_pass(srcs, dsts, g2a, zeros_tbl, T0, T1)
    tp2b = _sc_pass(srcs, dsts, g2b, zeros_tbl, T0, T1)
    g3a, g3b = _dense_layer2(tp2a, tp2b, g2a, g2b, dis, w2p, b2p)

    tp3a = _sc_pass(srcs, dsts, g3a, zeros_tbl, T0, T1)
    tp3b = _sc_pass(srcs, dsts, g3b, zeros_tbl, T0, T1)
    out = _dense_layer3(tp3a, tp3b, g3a, g3b, dis, w3p, b3p)
    return out[:_N]
